# baseline (speedup 1.0000x reference)
import functools
import jax
import jax.numpy as jnp
from jax import lax
from jax.experimental import pallas as pl
from jax.experimental.pallas import tpu as pltpu
from jax.experimental.pallas import tpu_sc as plsc

_NC, _NS = 2, 16
_NW = _NC * _NS


@functools.lru_cache(maxsize=None)
def _make_gather_max(R, D, Q, K):
    assert D % 16 == 0
    qpw = Q // _NW
    assert qpw * _NW == Q
    NB = 2
    tile_q = max(1, min(qpw, (65536 // NB) // (K * D)))
    while qpw % tile_q:
        tile_q -= 1
    n_sub = qpw // tile_q
    while n_sub % NB:
        assert tile_q % 2 == 0
        tile_q //= 2
        n_sub = qpw // tile_q

    mesh = plsc.VectorSubcoreMesh(core_axis_name="c", subcore_axis_name="s")

    @functools.partial(
        pl.kernel,
        out_type=jax.ShapeDtypeStruct((Q, D), jnp.float32),
        mesh=mesh,
        scratch_types=[
            pltpu.VMEM((qpw * K,), jnp.int32),
        ] + [pltpu.VMEM((tile_q * K, D), jnp.float32) for _ in range(NB)] + [
            pltpu.VMEM((qpw, D), jnp.float32),
        ] + [pltpu.SemaphoreType.DMA for _ in range(NB)],
        compiler_params=pltpu.CompilerParams(use_tc_tiling_on_sc=False),
    )
    def gather_max(table_hbm, idx_hbm, out_hbm, idx_v, *rest):
        bufs = rest[:NB]
        out_v = rest[NB]
        sems = rest[NB + 1:]
        wid = lax.axis_index("s") * _NC + lax.axis_index("c")
        base_q = wid * qpw
        pltpu.sync_copy(idx_hbm.at[pl.ds(base_q * K, qpw * K)], idx_v)

        def start(s, b):
            pltpu.async_copy(
                table_hbm.at[idx_v.at[pl.ds(s * (tile_q * K), tile_q * K)]],
                bufs[b], sems[b])

        def wait(b):
            pltpu.make_async_copy(
                table_hbm.at[idx_v.at[pl.ds(0, tile_q * K)]],
                bufs[b], sems[b]).wait()

        def compute(s, b):
            rows = bufs[b]

            def qbody(q, c2):
                for c in range(D // 16):
                    sl = pl.ds(c * 16, 16)
                    acc = rows[q * K, sl]
                    for k in range(1, K):
                        acc = jnp.maximum(acc, rows[q * K + k, sl])
                    out_v[s * tile_q + q, sl] = acc
                return c2

            lax.fori_loop(0, tile_q, qbody, 0, unroll=2)

        for b in range(NB - 1):
            start(b, b)

        def group(i, carry):
            s0 = i * NB
            for b in range(NB):
                start_s = s0 + b + (NB - 1)

                @pl.when(start_s < n_sub)
                def _(start_s=start_s, b=b):
                    start(start_s, (b + NB - 1) % NB)

                wait(b)
                compute(s0 + b, b)
            return carry

        lax.fori_loop(0, n_sub // NB, group, 0, unroll=False)
        pltpu.sync_copy(out_v, out_hbm.at[pl.ds(base_q, qpw)])

    return gather_max


def _gather_max(table, idx_flat, K):
    R, D = table.shape
    Q = idx_flat.shape[0] // K
    return _make_gather_max(R, D, Q, K)(table, idx_flat)


def _relu(x):
    return jnp.maximum(x, 0.0)


def _mm(a, w):
    return jnp.dot(a, w)


def _mmh(a, w):
    return jnp.dot(a, w, precision=jax.lax.Precision.HIGHEST)


def _qstn_pallas(pos, p):
    B, N, _ = pos.shape

    def mm(a, w):
        return jnp.dot(a, w, precision=jax.lax.Precision.HIGHEST)

    def body(pos_ref, w0, b0, w1, b1, w2, b2, w3, b3, w4, b4, w5, b5,
             t_ref, pt_ref):
        x = pos_ref[0]
        h = _relu(mm(x, w0[...]) + b0[...])
        h = _relu(mm(h, w1[...]) + b1[...])
        h = _relu(mm(h, w2[...]) + b2[...])
        v = jnp.max(h, axis=0, keepdims=True)
        v = _relu(mm(v, w3[...]) + b3[...])
        v = _relu(mm(v, w4[...]) + b4[...])
        q = mm(v, w5[...]) + b5[...]
        w, qx, qy, qz = (q[:, 0:1] + 1.0, q[:, 1:2], q[:, 2:3], q[:, 3:4])
        rn = jax.lax.rsqrt(w * w + qx * qx + qy * qy + qz * qz)
        w, qx, qy, qz = w * rn, qx * rn, qy * rn, qz * rn
        r = [1 - 2 * (qy * qy + qz * qz), 2 * (qx * qy - w * qz), 2 * (qx * qz + w * qy),
             2 * (qx * qy + w * qz), 1 - 2 * (qx * qx + qz * qz), 2 * (qy * qz - w * qx),
             2 * (qx * qz - w * qy), 2 * (qy * qz + w * qx), 1 - 2 * (qx * qx + qy * qy)]
        t_ref[0] = jnp.concatenate(r, axis=1)
        px, py, pz = x[:, 0:1], x[:, 1:2], x[:, 2:3]
        cols = [px * r[0] + py * r[3] + pz * r[6],
                px * r[1] + py * r[4] + pz * r[7],
                px * r[2] + py * r[5] + pz * r[8]]
        pt_ref[0] = jnp.concatenate(cols, axis=1)

    wspecs = []
    wargs = []
    for (w, b) in p:
        wspecs += [pl.BlockSpec(w.shape, lambda bb: (0, 0)),
                   pl.BlockSpec((1,) + b.shape, lambda bb: (0, 0))]
        wargs += [w, b.reshape(1, -1)]
    trans9, pos_t = pl.pallas_call(
        body,
        grid=(B,),
        in_specs=[pl.BlockSpec((1, N, 3), lambda bb: (bb, 0, 0))] + wspecs,
        out_specs=[pl.BlockSpec((1, 1, 9), lambda bb: (bb, 0, 0)),
                   pl.BlockSpec((1, N, 3), lambda bb: (bb, 0, 0))],
        out_shape=[jax.ShapeDtypeStruct((B, 1, 9), jnp.float32),
                   jax.ShapeDtypeStruct((B, N, 3), jnp.float32)],
    )(pos, *wargs)
    return trans9.reshape(B, 3, 3), pos_t


def _lfe_level_pallas(x1, x2, a1, a2, gm1, gm2, wc1, bc1, wc2, bc2):
    BN = x1.shape[0]

    def post(x, a, gm):
        if a is None:
            return x
        return jnp.concatenate([x, _relu(a + gm[:, :24])], axis=1)

    def post2(x, a, gm):
        if a is None:
            return x
        g = jnp.max(gm.reshape(gm.shape[0] // 2, 2, 32), axis=1)
        return jnp.concatenate([x, _relu(a + g[:, :24])], axis=1)

    def body(*refs):
        if a1 is None:
            x1r, x2r, wc1r, bc1r, wc2r, bc2r, x1o, x2o, a1o, a2o, tabo = refs
            x1n = x1r[...]
            x2n = x2r[...]
        else:
            (x1r, x2r, a1r, a2r, g1r, g2r, wc1r, bc1r, wc2r, bc2r,
             x1o, x2o, a1o, a2o, tabo) = refs
            x1n = post(x1r[...], a1r[...], g1r[...])
            x2n = post2(x2r[...], a2r[...], g2r[...])
        R = x1n.shape[0]
        ab1 = _mm(x1n, wc1r[...]) + bc1r[...]
        ab2 = _mm(x2n, wc2r[...]) + bc2r[...]
        x1o[...] = x1n
        x2o[...] = x2n
        a1o[...] = ab1[:, :24]
        a2o[...] = ab2[:, :24]
        z = jnp.zeros((R, 8), jnp.float32)
        tabo[0] = jnp.concatenate([ab1[:, 24:], z], axis=1)
        tabo[1] = jnp.concatenate([ab2[:, 24:], z], axis=1)

    C = x1.shape[1] + (0 if a1 is None else 24)
    G = 2
    R = BN // G

    def rows(a):
        return pl.BlockSpec((R, a.shape[1]), lambda b: (b, 0))

    def full(a):
        return pl.BlockSpec(a.shape, lambda b: (0, 0))

    if a1 is None:
        args = [x1, x2, wc1, bc1, wc2, bc2]
        in_specs = [rows(x1), rows(x2), full(wc1), full(bc1), full(wc2),
                    full(bc2)]
    else:
        args = [x1, x2, a1, a2, gm1, gm2, wc1, bc1, wc2, bc2]
        in_specs = [rows(x1), rows(x2), rows(a1), rows(a2), rows(gm1),
                    pl.BlockSpec((2 * R, 32), lambda b: (b, 0)),
                    full(wc1), full(bc1), full(wc2), full(bc2)]
    x1n, x2n, a1n, a2n, tab = pl.pallas_call(
        body,
        grid=(G,),
        in_specs=in_specs,
        out_specs=[pl.BlockSpec((R, C), lambda b: (b, 0)),
                   pl.BlockSpec((R, C), lambda b: (b, 0)),
                   pl.BlockSpec((R, 24), lambda b: (b, 0)),
                   pl.BlockSpec((R, 24), lambda b: (b, 0)),
                   pl.BlockSpec((2, R, 32), lambda b: (0, b, 0))],
        out_shape=[jax.ShapeDtypeStruct((BN, C), jnp.float32),
                   jax.ShapeDtypeStruct((BN, C), jnp.float32),
                   jax.ShapeDtypeStruct((BN, 24), jnp.float32),
                   jax.ShapeDtypeStruct((BN, 24), jnp.float32),
                   jax.ShapeDtypeStruct((2, BN, 32), jnp.float32)],
    )(*args)
    return x1n, x2n, a1n, a2n, tab.reshape(2 * BN, 32)


def _att_c12_pallas(x1, x2, a1, a2, gm1, gm2, watt, batt, wc1, bc1, wc2, bc2):
    BN = x1.shape[0]

    def body(x1r, x2r, a1r, a2r, g1r, g2r, war, bar, w1r, b1r, w2r, b2r, yo):
        y1 = jnp.concatenate([x1r[...], _relu(a1r[...] + g1r[...][:, :24])], axis=1)
        g2 = jnp.max(g2r[...].reshape(x2r.shape[0], 2, 32), axis=1)
        y2 = jnp.concatenate([x2r[...], _relu(a2r[...] + g2[:, :24])], axis=1)
        z = _mmh(y1 + y2, war[...]) + bar[...]
        s = 1.0 / (1.0 + jnp.exp(-z))
        y = s * y1 + (1.0 - s) * y2
        y = _relu(_mmh(y, w1r[...]) + b1r[...])
        yo[...] = _relu(_mmh(y, w2r[...]) + b2r[...])

    G = 2
    R = BN // G

    def rows(c):
        return pl.BlockSpec((R, c), lambda b: (b, 0))

    def full(a):
        return pl.BlockSpec(a.shape, lambda b: (0, 0))

    return pl.pallas_call(
        body,
        grid=(G,),
        in_specs=[rows(x1.shape[1]), rows(x2.shape[1]), rows(24), rows(24),
                  rows(32), pl.BlockSpec((2 * R, 32), lambda b: (b, 0)),
                  full(watt), full(batt), full(wc1),
                  full(bc1), full(wc2), full(bc2)],
        out_specs=rows(256),
        out_shape=jax.ShapeDtypeStruct((BN, 256), jnp.float32),
    )(x1, x2, a1, a2, gm1, gm2, watt, batt, wc1, bc1, wc2, bc2)


def _hier_pallas(yc, agg, gprev, w1, b1, w2, b2, nf, B):
    Qm = yc.shape[0]
    m = Qm // B

    def body(*refs):
        if gprev is None:
            ycr, aggr, w1r, b1r, w2r, b2r, yo, go = refs
        else:
            ycr, aggr, gpr, w1r, b1r, w2r, b2r, yo, go = refs
        ycv = ycr[...]
        a = aggr[...]
        if nf != 1:
            a = a - ycv
        f = jnp.concatenate([ycv, a], axis=1)
        if gprev is not None:
            gb = jnp.broadcast_to(gpr[...][:, None, :], (B, m, 128))
            f = jnp.concatenate([f, gb.reshape(Qm, 128)], axis=1)
        y_new = _relu(_mmh(f, w1r[...]) + b1r[...])
        yo[...] = y_new
        ymax = jnp.max(y_new.reshape(B, m, 256), axis=1)
        go[...] = _relu(_mmh(ymax, w2r[...]) + b2r[...])

    args = [yc, agg] + ([] if gprev is None else [gprev]) + [w1, b1, w2, b2]
    return pl.pallas_call(
        body,
        out_shape=[jax.ShapeDtypeStruct((Qm, 256), jnp.float32),
                   jax.ShapeDtypeStruct((B, 128), jnp.float32)],
    )(*args)


def _final_pallas(y, g1, g2, g3, g4, wc3, bc3, wc4, bc4, wcg, bcg,
                  wm1, bm1, wm2, bm2, B):
    Qm = y.shape[0]
    m = Qm // B

    def body(yr, g1r, g2r, g3r, g4r, w3r, b3r, w4r, b4r, wgr, bgr,
             wm1r, bm1r, wm2r, bm2r, yo, po):
        yv = yr[...]
        t = _relu(_mmh(yv, w3r[...]) + b3r[...]) + yv
        t = _relu(_mmh(t, w4r[...]) + b4r[...])
        yo[...] = t
        t3 = t.reshape(B, m, 128)[:, :64]
        t2 = t3.reshape(B * 64, 128)
        yg = _relu(_mmh(t2, wgr[...]) + bgr[...]) + t2
        y_g = jnp.max(yg.reshape(B, 64, 128), axis=1)
        h = jnp.concatenate([g1r[...], g2r[...], g3r[...], g4r[...], y_g],
                            axis=1)
        h = _relu(_mmh(h, wm1r[...]) + bm1r[...])
        po[...] = _relu(_mmh(h, wm2r[...]) + bm2r[...])

    return pl.pallas_call(
        body,
        out_shape=[jax.ShapeDtypeStruct((Qm, 128), jnp.float32),
                   jax.ShapeDtypeStruct((B, 128), jnp.float32)],
    )(y, g1, g2, g3, g4, wc3, bc3, wc4, bc4, wcg, bcg, wm1, bm1, wm2, bm2)


@functools.lru_cache(maxsize=None)
def _make_knn_search(B, M, n, k):
    def body(q_ref, p_ref, d_ref, v_ref):
        q = q_ref[0]
        p = p_ref[0]
        dx = q[:, 0:1] - p[:, 0][None, :]
        dy = q[:, 1:2] - p[:, 1][None, :]
        dz = q[:, 2:3] - p[:, 2][None, :]
        d = dx * dx + dy * dy + dz * dz
        ri = lax.broadcasted_iota(jnp.int32, (M, n), 0)
        ci = lax.broadcasted_iota(jnp.int32, (M, n), 1)
        d = jnp.where(ri == ci, jnp.float32(jnp.inf), d)
        di = lax.bitcast_convert_type(d, jnp.int32)

        def it(_, lohi):
            lo, hi = lohi
            mid = lo + (hi - lo) // 2
            cnt = jnp.sum((di <= mid).astype(jnp.float32), axis=1, keepdims=True)
            ge = cnt >= k
            return jnp.where(ge, lo, mid + 1), jnp.where(ge, mid, hi)

        lo0 = jnp.zeros((M, 1), jnp.int32)
        hi0 = jnp.full((M, 1), 0x7F800000, jnp.int32)
        lo, hi = lax.fori_loop(0, 31, it, (lo0, hi0))
        d_ref[0] = di
        v_ref[0, 0] = hi[:, 0]

    return pl.pallas_call(
        body,
        grid=(B,),
        in_specs=[
            pl.BlockSpec((1, M, 3), lambda b: (b, 0, 0)),
            pl.BlockSpec((1, n, 3), lambda b: (b, 0, 0)),
        ],
        out_specs=[
            pl.BlockSpec((1, M, n), lambda b: (b, 0, 0)),
            pl.BlockSpec((1, 1, M), lambda b: (b, 0, 0)),
        ],
        out_shape=[
            jax.ShapeDtypeStruct((B, M, n), jnp.int32),
            jax.ShapeDtypeStruct((B, 1, M), jnp.int32),
        ],
    )


@functools.lru_cache(maxsize=None)
def _make_knn_compact(B, M, n, k):
    Q = B * M
    qpt = Q // _NW
    tpb = _NW // B
    chunk_q = min(qpt, 65536 // n)
    while qpt % chunk_q:
        chunk_q -= 1
    n_chunks = qpt // chunk_q

    mesh = plsc.VectorSubcoreMesh(core_axis_name="c", subcore_axis_name="s")

    @functools.partial(
        pl.kernel,
        out_type=jax.ShapeDtypeStruct((Q * k,), jnp.int32),
        mesh=mesh,
        scratch_types=[
            pltpu.VMEM((chunk_q, n), jnp.int32),
            pltpu.VMEM((qpt,), jnp.int32),
            pltpu.VMEM((qpt * k + 16,), jnp.int32),
        ],
        compiler_params=pltpu.CompilerParams(use_tc_tiling_on_sc=False,
                                             needs_layout_passes=False),
    )
    def compact(d_hbm, v_hbm, out_hbm, d_v, v_v, out_v, ):
        wid = lax.axis_index("s") * _NC + lax.axis_index("c")
        base_q = wid * qpt
        boff = (wid // tpb) * n
        pltpu.sync_copy(v_hbm.at[pl.ds(base_q, qpt)], v_v)
        lane = jax.lax.iota(jnp.int32, 16)

        def chunk_body(c, carry):
            q0 = c * chunk_q
            pltpu.sync_copy(d_hbm.at[pl.ds(base_q + q0, chunk_q)], d_v)

            def q_body(q, carry2):
                vs = plsc.load_gather(v_v, [jnp.full((16,), 0, jnp.int32) + (q0 + q)])

                def j_body(j, cursor):
                    v = d_v[q, pl.ds(j * 16, 16)]
                    mask = v <= vs
                    cnt = plsc.cumsum(mask.astype(jnp.int32))
                    gpos = cnt + (cursor - 1 + (q0 + q) * k)
                    val = lane + (j * 16 + boff)
                    plsc.store_scatter(out_v, [gpos], val, mask=mask)
                    return cursor + jnp.sum(mask.astype(jnp.int32))

                lax.fori_loop(0, n // 16, j_body, jnp.int32(0), unroll=False)
                return carry2

            lax.fori_loop(0, chunk_q, q_body, 0, unroll=False)
            return carry

        lax.fori_loop(0, n_chunks, chunk_body, 0, unroll=False)
        pltpu.sync_copy(out_v.at[pl.ds(0, qpt * k)],
                        out_hbm.at[pl.ds(base_q * k, qpt * k)])

    return compact


def _knn_idx_flat(pos_q, pos_p, k):
    B, M, _ = pos_q.shape
    n = pos_p.shape[1]
    d_i32, vstar = _make_knn_search(B, M, n, k)(pos_q, pos_p)
    out = _make_knn_compact(B, M, n, k)(d_i32.reshape(B * M, n),
                                        vstar.reshape(B * M))
    return out


def kernel(pos, knn_idx, knn_idx_l, params):
    B, N, _ = pos.shape
    BN = B * N

    trans, pos_t = _qstn_pallas(pos, params["qstn"])
    pos = pos_t

    boffs = (jnp.arange(B, dtype=jnp.int32) * N)[:, None, None]
    idx_s = (knn_idx.astype(jnp.int32) + boffs)
    idx_l = (knn_idx_l.astype(jnp.int32) + boffs) + BN
    idx_lfe = jnp.concatenate(
        [idx_s.reshape(-1), idx_l.reshape(-1)], axis=0)

    def wcat(wb):
        w, b = wb
        C = w.shape[0] // 2
        return (jnp.concatenate([w[:C] - w[C:], w[C:]], axis=1),
                jnp.concatenate([b, jnp.zeros((24,), b.dtype)]).reshape(1, 48))

    x1 = pos.reshape(BN, 3)
    x2 = x1
    a1 = a2 = gm1 = gm2 = None
    for lvl in range(4):
        wc1, bc1 = wcat(params["enc1"][lvl])
        wc2, bc2 = wcat(params["enc2"][lvl])
        x1, x2, a1, a2, table = _lfe_level_pallas(
            x1, x2, a1, a2, gm1, gm2, wc1, bc1, wc2, bc2)
        gmax = _gather_max(table, idx_lfe, 16)
        gm1, gm2 = gmax[:BN], gmax[BN:]

    NUM_OUT = [512, 256, 128, 64]
    KNN_H1, KNN_H2 = 32, 16

    idx1 = _knn_idx_flat(pos[:, :NUM_OUT[0]], pos, KNN_H1)
    idx2 = _knn_idx_flat(pos[:, :NUM_OUT[1]], pos[:, :NUM_OUT[0]], KNN_H1)
    idx3 = _knn_idx_flat(pos[:, :NUM_OUT[2]], pos[:, :NUM_OUT[1]], KNN_H2)
    idx4 = _knn_idx_flat(pos[:, :NUM_OUT[2]], pos[:, :NUM_OUT[2]], KNN_H2)

    def b2d(b):
        return b.reshape(1, -1)

    y = _att_c12_pallas(x1, x2, a1, a2, gm1, gm2,
                        params["att"][0], b2d(params["att"][1]),
                        params["c1"][0], b2d(params["c1"][1]),
                        params["c2"][0], b2d(params["c2"][1]))

    def hier(y2d, idxf, m, p, g_prev, nf, K):
        agg = _gather_max(y2d, idxf, K)
        Nsrc = y2d.shape[0] // B
        yc = y2d.reshape(B, Nsrc, 256)[:, :m].reshape(B * m, 256)
        return _hier_pallas(yc, agg, g_prev,
                            p[0][0], b2d(p[0][1]), p[1][0], b2d(p[1][1]),
                            nf, B)

    y, g1 = hier(y, idx1, NUM_OUT[0], params["s1"], None, 1, KNN_H1)
    y, g2 = hier(y, idx2, NUM_OUT[1], params["s2"], g1, 2, KNN_H1)
    y, g3 = hier(y, idx3, NUM_OUT[2], params["s3"], g2, 1, KNN_H2)
    y, g4 = hier(y, idx4, NUM_OUT[2], params["s4"], g3, 2, KNN_H2)

    y_out, patch_global = _final_pallas(
        y, g1, g2, g3, g4,
        params["c3"][0], b2d(params["c3"][1]),
        params["c4"][0], b2d(params["c4"][1]),
        params["cg"][0], b2d(params["cg"][1]),
        params["mlp"][0][0], b2d(params["mlp"][0][1]),
        params["mlp"][1][0], b2d(params["mlp"][1][1]), B)

    y_out = jnp.transpose(y_out.reshape(B, NUM_OUT[2], 128), (0, 2, 1))
    return (y_out, trans, pos, patch_global)


# speedup vs baseline: 1.1273x; 1.1273x over previous
import functools
import jax
import jax.numpy as jnp
from jax import lax
from jax.experimental import pallas as pl
from jax.experimental.pallas import tpu as pltpu
from jax.experimental.pallas import tpu_sc as plsc

_NC, _NS = 2, 16
_NW = _NC * _NS


@functools.lru_cache(maxsize=None)
def _make_gather_max(R, D, Q, K):
    assert D % 16 == 0
    qpw = Q // _NW
    assert qpw * _NW == Q
    NB = 2
    tile_q = max(1, min(qpw, (65536 // NB) // (K * D)))
    while qpw % tile_q:
        tile_q -= 1
    n_sub = qpw // tile_q
    while n_sub % NB:
        assert tile_q % 2 == 0
        tile_q //= 2
        n_sub = qpw // tile_q

    mesh = plsc.VectorSubcoreMesh(core_axis_name="c", subcore_axis_name="s")

    @functools.partial(
        pl.kernel,
        out_type=jax.ShapeDtypeStruct((Q, D), jnp.float32),
        mesh=mesh,
        scratch_types=[
            pltpu.VMEM((qpw * K,), jnp.int32),
        ] + [pltpu.VMEM((tile_q * K, D), jnp.float32) for _ in range(NB)] + [
            pltpu.VMEM((qpw, D), jnp.float32),
        ] + [pltpu.SemaphoreType.DMA for _ in range(NB)],
        compiler_params=pltpu.CompilerParams(use_tc_tiling_on_sc=False),
    )
    def gather_max(table_hbm, idx_hbm, out_hbm, idx_v, *rest):
        bufs = rest[:NB]
        out_v = rest[NB]
        sems = rest[NB + 1:]
        wid = lax.axis_index("s") * _NC + lax.axis_index("c")
        base_q = wid * qpw
        pltpu.sync_copy(idx_hbm.at[pl.ds(base_q * K, qpw * K)], idx_v)

        def start(s, b):
            pltpu.async_copy(
                table_hbm.at[idx_v.at[pl.ds(s * (tile_q * K), tile_q * K)]],
                bufs[b], sems[b])

        def wait(b):
            pltpu.make_async_copy(
                table_hbm.at[idx_v.at[pl.ds(0, tile_q * K)]],
                bufs[b], sems[b]).wait()

        def compute(s, b):
            rows = bufs[b]

            def qbody(q, c2):
                for c in range(D // 16):
                    sl = pl.ds(c * 16, 16)
                    acc = rows[q * K, sl]
                    for k in range(1, K):
                        acc = jnp.maximum(acc, rows[q * K + k, sl])
                    out_v[s * tile_q + q, sl] = acc
                return c2

            lax.fori_loop(0, tile_q, qbody, 0, unroll=False)

        for b in range(NB - 1):
            start(b, b)

        def group(i, carry):
            s0 = i * NB
            for b in range(NB):
                start_s = s0 + b + (NB - 1)

                @pl.when(start_s < n_sub)
                def _(start_s=start_s, b=b):
                    start(start_s, (b + NB - 1) % NB)

                wait(b)
                compute(s0 + b, b)
            return carry

        lax.fori_loop(0, n_sub // NB, group, 0, unroll=False)
        pltpu.sync_copy(out_v, out_hbm.at[pl.ds(base_q, qpw)])

    return gather_max


def _gather_max(table, idx_flat, K):
    R, D = table.shape
    Q = idx_flat.shape[0] // K
    return _make_gather_max(R, D, Q, K)(table, idx_flat)


def _relu(x):
    return jnp.maximum(x, 0.0)


def _mm(a, w):
    return jnp.dot(a, w)


def _mmh(a, w):
    return jnp.dot(a, w, precision=jax.lax.Precision.HIGHEST)


def _qstn_pallas(pos, p):
    B, N, _ = pos.shape

    def mm(a, w):
        return jnp.dot(a, w, precision=jax.lax.Precision.HIGHEST)

    def body(pos_ref, w0, b0, w1, b1, w2, b2, w3, b3, w4, b4, w5, b5,
             t_ref, pt_ref):
        x = pos_ref[0]
        h = _relu(mm(x, w0[...]) + b0[...])
        h = _relu(mm(h, w1[...]) + b1[...])
        h = _relu(mm(h, w2[...]) + b2[...])
        v = jnp.max(h, axis=0, keepdims=True)
        v = _relu(mm(v, w3[...]) + b3[...])
        v = _relu(mm(v, w4[...]) + b4[...])
        q = mm(v, w5[...]) + b5[...]
        w, qx, qy, qz = (q[:, 0:1] + 1.0, q[:, 1:2], q[:, 2:3], q[:, 3:4])
        rn = jax.lax.rsqrt(w * w + qx * qx + qy * qy + qz * qz)
        w, qx, qy, qz = w * rn, qx * rn, qy * rn, qz * rn
        r = [1 - 2 * (qy * qy + qz * qz), 2 * (qx * qy - w * qz), 2 * (qx * qz + w * qy),
             2 * (qx * qy + w * qz), 1 - 2 * (qx * qx + qz * qz), 2 * (qy * qz - w * qx),
             2 * (qx * qz - w * qy), 2 * (qy * qz + w * qx), 1 - 2 * (qx * qx + qy * qy)]
        t_ref[0] = jnp.concatenate(r, axis=1)
        px, py, pz = x[:, 0:1], x[:, 1:2], x[:, 2:3]
        cols = [px * r[0] + py * r[3] + pz * r[6],
                px * r[1] + py * r[4] + pz * r[7],
                px * r[2] + py * r[5] + pz * r[8]]
        pt_ref[0] = jnp.concatenate(cols, axis=1)

    wspecs = []
    wargs = []
    for (w, b) in p:
        wspecs += [pl.BlockSpec(w.shape, lambda bb: (0, 0)),
                   pl.BlockSpec((1,) + b.shape, lambda bb: (0, 0))]
        wargs += [w, b.reshape(1, -1)]
    trans9, pos_t = pl.pallas_call(
        body,
        grid=(B,),
        in_specs=[pl.BlockSpec((1, N, 3), lambda bb: (bb, 0, 0))] + wspecs,
        out_specs=[pl.BlockSpec((1, 1, 9), lambda bb: (bb, 0, 0)),
                   pl.BlockSpec((1, N, 3), lambda bb: (bb, 0, 0))],
        out_shape=[jax.ShapeDtypeStruct((B, 1, 9), jnp.float32),
                   jax.ShapeDtypeStruct((B, N, 3), jnp.float32)],
    )(pos, *wargs)
    return trans9.reshape(B, 3, 3), pos_t


def _lfe_level_pallas(x1, x2, a1, a2, gm1, gm2, wc1, bc1, wc2, bc2):
    BN = x1.shape[0]

    def post(x, a, gm):
        if a is None:
            return x
        return jnp.concatenate([x, _relu(a + gm[:, :24])], axis=1)

    def post2(x, a, gm):
        if a is None:
            return x
        g = jnp.max(gm.reshape(gm.shape[0] // 2, 2, 32), axis=1)
        return jnp.concatenate([x, _relu(a + g[:, :24])], axis=1)

    def body(*refs):
        if a1 is None:
            x1r, x2r, wc1r, bc1r, wc2r, bc2r, x1o, x2o, a1o, a2o, tabo = refs
            x1n = x1r[...]
            x2n = x2r[...]
        else:
            (x1r, x2r, a1r, a2r, g1r, g2r, wc1r, bc1r, wc2r, bc2r,
             x1o, x2o, a1o, a2o, tabo) = refs
            x1n = post(x1r[...], a1r[...], g1r[...])
            x2n = post(x2r[...], a2r[...], g2r[...])
        R = x1n.shape[0]
        ab1 = _mm(x1n, wc1r[...]) + bc1r[...]
        ab2 = _mm(x2n, wc2r[...]) + bc2r[...]
        x1o[...] = x1n
        x2o[...] = x2n
        a1o[...] = ab1[:, :24]
        a2o[...] = ab2[:, :24]
        z = jnp.zeros((R, 8), jnp.float32)
        tabo[0] = jnp.concatenate([ab1[:, 24:], z], axis=1)
        tabo[1] = jnp.concatenate([ab2[:, 24:], z], axis=1)

    C = x1.shape[1] + (0 if a1 is None else 24)
    G = 2
    R = BN // G

    def rows(a):
        return pl.BlockSpec((R, a.shape[1]), lambda b: (b, 0))

    def full(a):
        return pl.BlockSpec(a.shape, lambda b: (0, 0))

    if a1 is None:
        args = [x1, x2, wc1, bc1, wc2, bc2]
        in_specs = [rows(x1), rows(x2), full(wc1), full(bc1), full(wc2),
                    full(bc2)]
    else:
        args = [x1, x2, a1, a2, gm1, gm2, wc1, bc1, wc2, bc2]
        in_specs = [rows(x1), rows(x2), rows(a1), rows(a2), rows(gm1),
                    rows(gm2), full(wc1), full(bc1), full(wc2), full(bc2)]
    x1n, x2n, a1n, a2n, tab = pl.pallas_call(
        body,
        grid=(G,),
        in_specs=in_specs,
        out_specs=[pl.BlockSpec((R, C), lambda b: (b, 0)),
                   pl.BlockSpec((R, C), lambda b: (b, 0)),
                   pl.BlockSpec((R, 24), lambda b: (b, 0)),
                   pl.BlockSpec((R, 24), lambda b: (b, 0)),
                   pl.BlockSpec((2, R, 32), lambda b: (0, b, 0))],
        out_shape=[jax.ShapeDtypeStruct((BN, C), jnp.float32),
                   jax.ShapeDtypeStruct((BN, C), jnp.float32),
                   jax.ShapeDtypeStruct((BN, 24), jnp.float32),
                   jax.ShapeDtypeStruct((BN, 24), jnp.float32),
                   jax.ShapeDtypeStruct((2, BN, 32), jnp.float32)],
    )(*args)
    return x1n, x2n, a1n, a2n, tab.reshape(2 * BN, 32)


def _att_c12_pallas(x1, x2, a1, a2, gm1, gm2, watt, batt, wc1, bc1, wc2, bc2):
    BN = x1.shape[0]

    def body(x1r, x2r, a1r, a2r, g1r, g2r, war, bar, w1r, b1r, w2r, b2r, yo):
        y1 = jnp.concatenate([x1r[...], _relu(a1r[...] + g1r[...][:, :24])], axis=1)
        y2 = jnp.concatenate([x2r[...], _relu(a2r[...] + g2r[...][:, :24])], axis=1)
        z = _mmh(y1 + y2, war[...]) + bar[...]
        s = 1.0 / (1.0 + jnp.exp(-z))
        y = s * y1 + (1.0 - s) * y2
        y = _relu(_mmh(y, w1r[...]) + b1r[...])
        yo[...] = _relu(_mmh(y, w2r[...]) + b2r[...])

    G = 2
    R = BN // G

    def rows(c):
        return pl.BlockSpec((R, c), lambda b: (b, 0))

    def full(a):
        return pl.BlockSpec(a.shape, lambda b: (0, 0))

    return pl.pallas_call(
        body,
        grid=(G,),
        in_specs=[rows(x1.shape[1]), rows(x2.shape[1]), rows(24), rows(24),
                  rows(32), rows(32), full(watt), full(batt), full(wc1),
                  full(bc1), full(wc2), full(bc2)],
        out_specs=rows(256),
        out_shape=jax.ShapeDtypeStruct((BN, 256), jnp.float32),
    )(x1, x2, a1, a2, gm1, gm2, watt, batt, wc1, bc1, wc2, bc2)


def _hier_pallas(yc, agg, gprev, w1, b1, w2, b2, nf, B):
    Qm = yc.shape[0]
    m = Qm // B

    def body(*refs):
        if gprev is None:
            ycr, aggr, w1r, b1r, w2r, b2r, yo, go = refs
        else:
            ycr, aggr, gpr, w1r, b1r, w2r, b2r, yo, go = refs
        ycv = ycr[...]
        a = aggr[...]
        if nf != 1:
            a = a - ycv
        f = jnp.concatenate([ycv, a], axis=1)
        if gprev is not None:
            gb = jnp.broadcast_to(gpr[...][:, None, :], (B, m, 128))
            f = jnp.concatenate([f, gb.reshape(Qm, 128)], axis=1)
        y_new = _relu(_mmh(f, w1r[...]) + b1r[...])
        yo[...] = y_new
        ymax = jnp.max(y_new.reshape(B, m, 256), axis=1)
        go[...] = _relu(_mmh(ymax, w2r[...]) + b2r[...])

    args = [yc, agg] + ([] if gprev is None else [gprev]) + [w1, b1, w2, b2]
    return pl.pallas_call(
        body,
        out_shape=[jax.ShapeDtypeStruct((Qm, 256), jnp.float32),
                   jax.ShapeDtypeStruct((B, 128), jnp.float32)],
    )(*args)


def _final_pallas(y, g1, g2, g3, g4, wc3, bc3, wc4, bc4, wcg, bcg,
                  wm1, bm1, wm2, bm2, B):
    Qm = y.shape[0]
    m = Qm // B

    def body(yr, g1r, g2r, g3r, g4r, w3r, b3r, w4r, b4r, wgr, bgr,
             wm1r, bm1r, wm2r, bm2r, yo, po):
        yv = yr[...]
        t = _relu(_mmh(yv, w3r[...]) + b3r[...]) + yv
        t = _relu(_mmh(t, w4r[...]) + b4r[...])
        yo[...] = t
        t3 = t.reshape(B, m, 128)[:, :64]
        t2 = t3.reshape(B * 64, 128)
        yg = _relu(_mmh(t2, wgr[...]) + bgr[...]) + t2
        y_g = jnp.max(yg.reshape(B, 64, 128), axis=1)
        h = jnp.concatenate([g1r[...], g2r[...], g3r[...], g4r[...], y_g],
                            axis=1)
        h = _relu(_mmh(h, wm1r[...]) + bm1r[...])
        po[...] = _relu(_mmh(h, wm2r[...]) + bm2r[...])

    return pl.pallas_call(
        body,
        out_shape=[jax.ShapeDtypeStruct((Qm, 128), jnp.float32),
                   jax.ShapeDtypeStruct((B, 128), jnp.float32)],
    )(y, g1, g2, g3, g4, wc3, bc3, wc4, bc4, wcg, bcg, wm1, bm1, wm2, bm2)


@functools.lru_cache(maxsize=None)
def _make_knn_search(B, M, n, k):
    def body(q_ref, p_ref, d_ref, v_ref):
        q = q_ref[0]
        p = p_ref[0]
        dx = q[:, 0:1] - p[:, 0][None, :]
        dy = q[:, 1:2] - p[:, 1][None, :]
        dz = q[:, 2:3] - p[:, 2][None, :]
        d = dx * dx + dy * dy + dz * dz
        ri = lax.broadcasted_iota(jnp.int32, (M, n), 0)
        ci = lax.broadcasted_iota(jnp.int32, (M, n), 1)
        d = jnp.where(ri == ci, jnp.float32(jnp.inf), d)
        di = lax.bitcast_convert_type(d, jnp.int32)

        def it(_, lohi):
            lo, hi = lohi
            mid = lo + (hi - lo) // 2
            cnt = jnp.sum((di <= mid).astype(jnp.float32), axis=1, keepdims=True)
            ge = cnt >= k
            return jnp.where(ge, lo, mid + 1), jnp.where(ge, mid, hi)

        lo0 = jnp.zeros((M, 1), jnp.int32)
        hi0 = jnp.full((M, 1), 0x7F800000, jnp.int32)
        lo, hi = lax.fori_loop(0, 31, it, (lo0, hi0))
        d_ref[0] = di
        v_ref[0, 0] = hi[:, 0]

    return pl.pallas_call(
        body,
        grid=(B,),
        in_specs=[
            pl.BlockSpec((1, M, 3), lambda b: (b, 0, 0)),
            pl.BlockSpec((1, n, 3), lambda b: (b, 0, 0)),
        ],
        out_specs=[
            pl.BlockSpec((1, M, n), lambda b: (b, 0, 0)),
            pl.BlockSpec((1, 1, M), lambda b: (b, 0, 0)),
        ],
        out_shape=[
            jax.ShapeDtypeStruct((B, M, n), jnp.int32),
            jax.ShapeDtypeStruct((B, 1, M), jnp.int32),
        ],
    )


@functools.lru_cache(maxsize=None)
def _make_knn_compact(B, M, n, k):
    Q = B * M
    qpt = Q // _NW
    tpb = _NW // B
    chunk_q = min(qpt, 65536 // n)
    while qpt % chunk_q:
        chunk_q -= 1
    n_chunks = qpt // chunk_q

    mesh = plsc.VectorSubcoreMesh(core_axis_name="c", subcore_axis_name="s")

    @functools.partial(
        pl.kernel,
        out_type=jax.ShapeDtypeStruct((Q * k,), jnp.int32),
        mesh=mesh,
        scratch_types=[
            pltpu.VMEM((chunk_q, n), jnp.int32),
            pltpu.VMEM((qpt,), jnp.int32),
            pltpu.VMEM((qpt * k + 16,), jnp.int32),
        ],
        compiler_params=pltpu.CompilerParams(use_tc_tiling_on_sc=False,
                                             needs_layout_passes=False),
    )
    def compact(d_hbm, v_hbm, out_hbm, d_v, v_v, out_v, ):
        wid = lax.axis_index("s") * _NC + lax.axis_index("c")
        base_q = wid * qpt
        boff = (wid // tpb) * n
        pltpu.sync_copy(v_hbm.at[pl.ds(base_q, qpt)], v_v)
        lane = jax.lax.iota(jnp.int32, 16)

        def chunk_body(c, carry):
            q0 = c * chunk_q
            pltpu.sync_copy(d_hbm.at[pl.ds(base_q + q0, chunk_q)], d_v)

            def q_body(q, carry2):
                vs = plsc.load_gather(v_v, [jnp.full((16,), 0, jnp.int32) + (q0 + q)])

                def j_body(j, cursor):
                    v = d_v[q, pl.ds(j * 16, 16)]
                    mask = v <= vs
                    cnt = plsc.cumsum(mask.astype(jnp.int32))
                    gpos = cnt + (cursor - 1 + (q0 + q) * k)
                    val = lane + (j * 16 + boff)
                    plsc.store_scatter(out_v, [gpos], val, mask=mask)
                    return cursor + jnp.sum(mask.astype(jnp.int32))

                lax.fori_loop(0, n // 16, j_body, jnp.int32(0), unroll=False)
                return carry2

            lax.fori_loop(0, chunk_q, q_body, 0, unroll=False)
            return carry

        lax.fori_loop(0, n_chunks, chunk_body, 0, unroll=False)
        pltpu.sync_copy(out_v.at[pl.ds(0, qpt * k)],
                        out_hbm.at[pl.ds(base_q * k, qpt * k)])

    return compact


def _knn_idx_flat(pos_q, pos_p, k):
    B, M, _ = pos_q.shape
    n = pos_p.shape[1]
    d_i32, vstar = _make_knn_search(B, M, n, k)(pos_q, pos_p)
    out = _make_knn_compact(B, M, n, k)(d_i32.reshape(B * M, n),
                                        vstar.reshape(B * M))
    return out


def kernel(pos, knn_idx, knn_idx_l, params):
    B, N, _ = pos.shape
    BN = B * N

    trans, pos_t = _qstn_pallas(pos, params["qstn"])
    pos = pos_t

    boffs = (jnp.arange(B, dtype=jnp.int32) * N)[:, None, None]
    idx_s = (knn_idx.astype(jnp.int32) + boffs)
    idx_s = jnp.concatenate([idx_s, idx_s], axis=-1)
    idx_l = (knn_idx_l.astype(jnp.int32) + boffs) + BN
    idx_lfe = jnp.concatenate(
        [idx_s.reshape(-1), idx_l.reshape(-1)], axis=0)

    def wcat(wb):
        w, b = wb
        C = w.shape[0] // 2
        return (jnp.concatenate([w[:C] - w[C:], w[C:]], axis=1),
                jnp.concatenate([b, jnp.zeros((24,), b.dtype)]).reshape(1, 48))

    x1 = pos.reshape(BN, 3)
    x2 = x1
    a1 = a2 = gm1 = gm2 = None
    for lvl in range(4):
        wc1, bc1 = wcat(params["enc1"][lvl])
        wc2, bc2 = wcat(params["enc2"][lvl])
        x1, x2, a1, a2, table = _lfe_level_pallas(
            x1, x2, a1, a2, gm1, gm2, wc1, bc1, wc2, bc2)
        gmax = _gather_max(table, idx_lfe, 32)
        gm1, gm2 = gmax[:BN], gmax[BN:]

    NUM_OUT = [512, 256, 128, 64]
    KNN_H1, KNN_H2 = 32, 16

    idx1 = _knn_idx_flat(pos[:, :NUM_OUT[0]], pos, KNN_H1)
    idx2 = _knn_idx_flat(pos[:, :NUM_OUT[1]], pos[:, :NUM_OUT[0]], KNN_H1)
    idx3 = _knn_idx_flat(pos[:, :NUM_OUT[2]], pos[:, :NUM_OUT[1]], KNN_H2)
    idx4 = _knn_idx_flat(pos[:, :NUM_OUT[2]], pos[:, :NUM_OUT[2]], KNN_H2)

    def b2d(b):
        return b.reshape(1, -1)

    y = _att_c12_pallas(x1, x2, a1, a2, gm1, gm2,
                        params["att"][0], b2d(params["att"][1]),
                        params["c1"][0], b2d(params["c1"][1]),
                        params["c2"][0], b2d(params["c2"][1]))

    def hier(y2d, idxf, m, p, g_prev, nf, K):
        agg = _gather_max(y2d, idxf, K)
        Nsrc = y2d.shape[0] // B
        yc = y2d.reshape(B, Nsrc, 256)[:, :m].reshape(B * m, 256)
        return _hier_pallas(yc, agg, g_prev,
                            p[0][0], b2d(p[0][1]), p[1][0], b2d(p[1][1]),
                            nf, B)

    y, g1 = hier(y, idx1, NUM_OUT[0], params["s1"], None, 1, KNN_H1)
    y, g2 = hier(y, idx2, NUM_OUT[1], params["s2"], g1, 2, KNN_H1)
    y, g3 = hier(y, idx3, NUM_OUT[2], params["s3"], g2, 1, KNN_H2)
    y, g4 = hier(y, idx4, NUM_OUT[2], params["s4"], g3, 2, KNN_H2)

    y_out, patch_global = _final_pallas(
        y, g1, g2, g3, g4,
        params["c3"][0], b2d(params["c3"][1]),
        params["c4"][0], b2d(params["c4"][1]),
        params["cg"][0], b2d(params["cg"][1]),
        params["mlp"][0][0], b2d(params["mlp"][0][1]),
        params["mlp"][1][0], b2d(params["mlp"][1][1]), B)

    y_out = jnp.transpose(y_out.reshape(B, NUM_OUT[2], 128), (0, 2, 1))
    return (y_out, trans, pos, patch_global)


# speedup vs baseline: 1.1274x; 1.0001x over previous
import functools
import jax
import jax.numpy as jnp
from jax import lax
from jax.experimental import pallas as pl
from jax.experimental.pallas import tpu as pltpu
from jax.experimental.pallas import tpu_sc as plsc

_NC, _NS = 2, 16
_NW = _NC * _NS


@functools.lru_cache(maxsize=None)
def _make_gather_max(R, D, Q, K):
    assert D % 16 == 0
    qpw = Q // _NW
    assert qpw * _NW == Q
    NB = 2
    tile_q = max(1, min(qpw, (65536 // NB) // (K * D)))
    while qpw % tile_q:
        tile_q -= 1
    n_sub = qpw // tile_q
    while n_sub % NB:
        assert tile_q % 2 == 0
        tile_q //= 2
        n_sub = qpw // tile_q

    mesh = plsc.VectorSubcoreMesh(core_axis_name="c", subcore_axis_name="s")

    @functools.partial(
        pl.kernel,
        out_type=jax.ShapeDtypeStruct((Q, D), jnp.float32),
        mesh=mesh,
        scratch_types=[
            pltpu.VMEM((qpw * K,), jnp.int32),
        ] + [pltpu.VMEM((tile_q * K, D), jnp.float32) for _ in range(NB)] + [
            pltpu.VMEM((qpw, D), jnp.float32),
        ] + [pltpu.SemaphoreType.DMA for _ in range(NB)],
        compiler_params=pltpu.CompilerParams(use_tc_tiling_on_sc=False),
    )
    def gather_max(table_hbm, idx_hbm, out_hbm, idx_v, *rest):
        bufs = rest[:NB]
        out_v = rest[NB]
        sems = rest[NB + 1:]
        wid = lax.axis_index("s") * _NC + lax.axis_index("c")
        base_q = wid * qpw
        pltpu.sync_copy(idx_hbm.at[pl.ds(base_q * K, qpw * K)], idx_v)

        def start(s, b):
            pltpu.async_copy(
                table_hbm.at[idx_v.at[pl.ds(s * (tile_q * K), tile_q * K)]],
                bufs[b], sems[b])

        def wait(b):
            pltpu.make_async_copy(
                table_hbm.at[idx_v.at[pl.ds(0, tile_q * K)]],
                bufs[b], sems[b]).wait()

        def compute(s, b):
            rows = bufs[b]

            def qbody(q, c2):
                for c in range(D // 16):
                    sl = pl.ds(c * 16, 16)
                    acc = rows[q * K, sl]
                    for k in range(1, K):
                        acc = jnp.maximum(acc, rows[q * K + k, sl])
                    out_v[s * tile_q + q, sl] = acc
                return c2

            lax.fori_loop(0, tile_q, qbody, 0, unroll=False)

        for b in range(NB - 1):
            start(b, b)

        def group(i, carry):
            s0 = i * NB
            for b in range(NB):
                start_s = s0 + b + (NB - 1)

                @pl.when(start_s < n_sub)
                def _(start_s=start_s, b=b):
                    start(start_s, (b + NB - 1) % NB)

                wait(b)
                compute(s0 + b, b)
            return carry

        lax.fori_loop(0, n_sub // NB, group, 0, unroll=False)
        pltpu.sync_copy(out_v, out_hbm.at[pl.ds(base_q, qpw)])

    return gather_max


def _gather_max(table, idx_flat, K):
    R, D = table.shape
    Q = idx_flat.shape[0] // K
    return _make_gather_max(R, D, Q, K)(table, idx_flat)


def _relu(x):
    return jnp.maximum(x, 0.0)


def _mm(a, w):
    return jnp.dot(a, w)


def _mmh(a, w):
    return jnp.dot(a, w, precision=jax.lax.Precision.HIGHEST)


def _qstn_pallas(pos, p):
    B, N, _ = pos.shape

    def mm(a, w):
        return jnp.dot(a, w, precision=jax.lax.Precision.HIGHEST)

    def body(pos_ref, w0, b0, w1, b1, w2, b2, w3, b3, w4, b4, w5, b5,
             t_ref, pt_ref):
        x = pos_ref[0]
        h = _relu(mm(x, w0[...]) + b0[...])
        h = _relu(mm(h, w1[...]) + b1[...])
        h = _relu(mm(h, w2[...]) + b2[...])
        v = jnp.max(h, axis=0, keepdims=True)
        v = _relu(mm(v, w3[...]) + b3[...])
        v = _relu(mm(v, w4[...]) + b4[...])
        q = mm(v, w5[...]) + b5[...]
        w, qx, qy, qz = (q[:, 0:1] + 1.0, q[:, 1:2], q[:, 2:3], q[:, 3:4])
        rn = jax.lax.rsqrt(w * w + qx * qx + qy * qy + qz * qz)
        w, qx, qy, qz = w * rn, qx * rn, qy * rn, qz * rn
        r = [1 - 2 * (qy * qy + qz * qz), 2 * (qx * qy - w * qz), 2 * (qx * qz + w * qy),
             2 * (qx * qy + w * qz), 1 - 2 * (qx * qx + qz * qz), 2 * (qy * qz - w * qx),
             2 * (qx * qz - w * qy), 2 * (qy * qz + w * qx), 1 - 2 * (qx * qx + qy * qy)]
        t_ref[0] = jnp.concatenate(r, axis=1)
        px, py, pz = x[:, 0:1], x[:, 1:2], x[:, 2:3]
        cols = [px * r[0] + py * r[3] + pz * r[6],
                px * r[1] + py * r[4] + pz * r[7],
                px * r[2] + py * r[5] + pz * r[8]]
        pt_ref[0] = jnp.concatenate(cols, axis=1)

    wspecs = []
    wargs = []
    for (w, b) in p:
        wspecs += [pl.BlockSpec(w.shape, lambda bb: (0, 0)),
                   pl.BlockSpec((1,) + b.shape, lambda bb: (0, 0))]
        wargs += [w, b.reshape(1, -1)]
    trans9, pos_t = pl.pallas_call(
        body,
        grid=(B,),
        in_specs=[pl.BlockSpec((1, N, 3), lambda bb: (bb, 0, 0))] + wspecs,
        out_specs=[pl.BlockSpec((1, 1, 9), lambda bb: (bb, 0, 0)),
                   pl.BlockSpec((1, N, 3), lambda bb: (bb, 0, 0))],
        out_shape=[jax.ShapeDtypeStruct((B, 1, 9), jnp.float32),
                   jax.ShapeDtypeStruct((B, N, 3), jnp.float32)],
    )(pos, *wargs)
    return trans9.reshape(B, 3, 3), pos_t


def _lfe_level_pallas(x1, x2, a1, a2, gm1, gm2, wc1, bc1, wc2, bc2):
    BN = x1.shape[0]

    def post(x, a, gm):
        if a is None:
            return x
        return jnp.concatenate([x, _relu(a + gm[:, :24])], axis=1)

    def post2(x, a, gm):
        if a is None:
            return x
        g = jnp.max(gm.reshape(gm.shape[0] // 2, 2, 32), axis=1)
        return jnp.concatenate([x, _relu(a + g[:, :24])], axis=1)

    def body(*refs):
        if a1 is None:
            x1r, x2r, wc1r, bc1r, wc2r, bc2r, x1o, x2o, a1o, a2o, tabo = refs
            x1n = x1r[...]
            x2n = x2r[...]
        else:
            (x1r, x2r, a1r, a2r, g1r, g2r, wc1r, bc1r, wc2r, bc2r,
             x1o, x2o, a1o, a2o, tabo) = refs
            x1n = post(x1r[...], a1r[...], g1r[...])
            x2n = post(x2r[...], a2r[...], g2r[...])
        R = x1n.shape[0]
        ab1 = _mm(x1n, wc1r[...]) + bc1r[...]
        ab2 = _mm(x2n, wc2r[...]) + bc2r[...]
        x1o[...] = x1n
        x2o[...] = x2n
        a1o[...] = ab1[:, :24]
        a2o[...] = ab2[:, :24]
        z = jnp.zeros((R, 8), jnp.float32)
        tabo[0] = jnp.concatenate([ab1[:, 24:], z], axis=1)
        tabo[1] = jnp.concatenate([ab2[:, 24:], z], axis=1)

    C = x1.shape[1] + (0 if a1 is None else 24)
    G = 2
    R = BN // G

    def rows(a):
        return pl.BlockSpec((R, a.shape[1]), lambda b: (b, 0))

    def full(a):
        return pl.BlockSpec(a.shape, lambda b: (0, 0))

    if a1 is None:
        args = [x1, x2, wc1, bc1, wc2, bc2]
        in_specs = [rows(x1), rows(x2), full(wc1), full(bc1), full(wc2),
                    full(bc2)]
    else:
        args = [x1, x2, a1, a2, gm1, gm2, wc1, bc1, wc2, bc2]
        in_specs = [rows(x1), rows(x2), rows(a1), rows(a2), rows(gm1),
                    rows(gm2), full(wc1), full(bc1), full(wc2), full(bc2)]
    x1n, x2n, a1n, a2n, tab = pl.pallas_call(
        body,
        grid=(G,),
        in_specs=in_specs,
        out_specs=[pl.BlockSpec((R, C), lambda b: (b, 0)),
                   pl.BlockSpec((R, C), lambda b: (b, 0)),
                   pl.BlockSpec((R, 24), lambda b: (b, 0)),
                   pl.BlockSpec((R, 24), lambda b: (b, 0)),
                   pl.BlockSpec((2, R, 32), lambda b: (0, b, 0))],
        out_shape=[jax.ShapeDtypeStruct((BN, C), jnp.float32),
                   jax.ShapeDtypeStruct((BN, C), jnp.float32),
                   jax.ShapeDtypeStruct((BN, 24), jnp.float32),
                   jax.ShapeDtypeStruct((BN, 24), jnp.float32),
                   jax.ShapeDtypeStruct((2, BN, 32), jnp.float32)],
    )(*args)
    return x1n, x2n, a1n, a2n, tab.reshape(2 * BN, 32)


def _att_c12_pallas(x1, x2, a1, a2, gm1, gm2, watt, batt, wc1, bc1, wc2, bc2):
    BN = x1.shape[0]

    def body(x1r, x2r, a1r, a2r, g1r, g2r, war, bar, w1r, b1r, w2r, b2r, yo):
        y1 = jnp.concatenate([x1r[...], _relu(a1r[...] + g1r[...][:, :24])], axis=1)
        y2 = jnp.concatenate([x2r[...], _relu(a2r[...] + g2r[...][:, :24])], axis=1)
        z = _mmh(y1 + y2, war[...]) + bar[...]
        s = 1.0 / (1.0 + jnp.exp(-z))
        y = s * y1 + (1.0 - s) * y2
        y = _relu(_mmh(y, w1r[...]) + b1r[...])
        yo[...] = _relu(_mmh(y, w2r[...]) + b2r[...])

    G = 2
    R = BN // G

    def rows(c):
        return pl.BlockSpec((R, c), lambda b: (b, 0))

    def full(a):
        return pl.BlockSpec(a.shape, lambda b: (0, 0))

    return pl.pallas_call(
        body,
        grid=(G,),
        in_specs=[rows(x1.shape[1]), rows(x2.shape[1]), rows(24), rows(24),
                  rows(32), rows(32), full(watt), full(batt), full(wc1),
                  full(bc1), full(wc2), full(bc2)],
        out_specs=rows(256),
        out_shape=jax.ShapeDtypeStruct((BN, 256), jnp.float32),
    )(x1, x2, a1, a2, gm1, gm2, watt, batt, wc1, bc1, wc2, bc2)


def _hier_pallas(yc, agg, gprev, w1, b1, w2, b2, nf, B):
    Qm = yc.shape[0]
    m = Qm // B

    def body(*refs):
        if gprev is None:
            ycr, aggr, w1r, b1r, w2r, b2r, yo, go = refs
        else:
            ycr, aggr, gpr, w1r, b1r, w2r, b2r, yo, go = refs
        ycv = ycr[...]
        a = aggr[...]
        if nf != 1:
            a = a - ycv
        f = jnp.concatenate([ycv, a], axis=1)
        if gprev is not None:
            gb = jnp.broadcast_to(gpr[...][:, None, :], (B, m, 128))
            f = jnp.concatenate([f, gb.reshape(Qm, 128)], axis=1)
        y_new = _relu(_mmh(f, w1r[...]) + b1r[...])
        yo[...] = y_new
        ymax = jnp.max(y_new.reshape(B, m, 256), axis=1)
        go[...] = _relu(_mmh(ymax, w2r[...]) + b2r[...])

    args = [yc, agg] + ([] if gprev is None else [gprev]) + [w1, b1, w2, b2]
    return pl.pallas_call(
        body,
        out_shape=[jax.ShapeDtypeStruct((Qm, 256), jnp.float32),
                   jax.ShapeDtypeStruct((B, 128), jnp.float32)],
    )(*args)


def _final_pallas(y, g1, g2, g3, g4, wc3, bc3, wc4, bc4, wcg, bcg,
                  wm1, bm1, wm2, bm2, B):
    Qm = y.shape[0]
    m = Qm // B

    def body(yr, g1r, g2r, g3r, g4r, w3r, b3r, w4r, b4r, wgr, bgr,
             wm1r, bm1r, wm2r, bm2r, yo, po):
        yv = yr[...]
        t = _relu(_mmh(yv, w3r[...]) + b3r[...]) + yv
        t = _relu(_mmh(t, w4r[...]) + b4r[...])
        yo[...] = t
        t3 = t.reshape(B, m, 128)[:, :64]
        t2 = t3.reshape(B * 64, 128)
        yg = _relu(_mmh(t2, wgr[...]) + bgr[...]) + t2
        y_g = jnp.max(yg.reshape(B, 64, 128), axis=1)
        h = jnp.concatenate([g1r[...], g2r[...], g3r[...], g4r[...], y_g],
                            axis=1)
        h = _relu(_mmh(h, wm1r[...]) + bm1r[...])
        po[...] = _relu(_mmh(h, wm2r[...]) + bm2r[...])

    return pl.pallas_call(
        body,
        out_shape=[jax.ShapeDtypeStruct((Qm, 128), jnp.float32),
                   jax.ShapeDtypeStruct((B, 128), jnp.float32)],
    )(y, g1, g2, g3, g4, wc3, bc3, wc4, bc4, wcg, bcg, wm1, bm1, wm2, bm2)


@functools.lru_cache(maxsize=None)
def _make_knn_search(B, M, n, k):
    def body(q_ref, p_ref, d_ref, v_ref):
        q = q_ref[0]
        p = p_ref[0]
        dx = q[:, 0:1] - p[:, 0][None, :]
        dy = q[:, 1:2] - p[:, 1][None, :]
        dz = q[:, 2:3] - p[:, 2][None, :]
        d = dx * dx + dy * dy + dz * dz
        ri = lax.broadcasted_iota(jnp.int32, (M, n), 0)
        ci = lax.broadcasted_iota(jnp.int32, (M, n), 1)
        d = jnp.where(ri == ci, jnp.float32(jnp.inf), d)
        di = lax.bitcast_convert_type(d, jnp.int32)

        def it(_, lohi):
            lo, hi = lohi
            mid = lo + (hi - lo) // 2
            cnt = jnp.sum((di <= mid).astype(jnp.float32), axis=1, keepdims=True)
            ge = cnt >= k
            return jnp.where(ge, lo, mid + 1), jnp.where(ge, mid, hi)

        lo0 = jnp.zeros((M, 1), jnp.int32)
        hi0 = jnp.full((M, 1), 0x7F800000, jnp.int32)
        lo, hi = lax.fori_loop(0, 31, it, (lo0, hi0))
        d_ref[0] = di
        v_ref[0, 0] = hi[:, 0]

    return pl.pallas_call(
        body,
        grid=(B,),
        in_specs=[
            pl.BlockSpec((1, M, 3), lambda b: (b, 0, 0)),
            pl.BlockSpec((1, n, 3), lambda b: (b, 0, 0)),
        ],
        out_specs=[
            pl.BlockSpec((1, M, n), lambda b: (b, 0, 0)),
            pl.BlockSpec((1, 1, M), lambda b: (b, 0, 0)),
        ],
        out_shape=[
            jax.ShapeDtypeStruct((B, M, n), jnp.int32),
            jax.ShapeDtypeStruct((B, 1, M), jnp.int32),
        ],
    )


@functools.lru_cache(maxsize=None)
def _make_knn_compact(B, M, n, k):
    Q = B * M
    qpt = Q // _NW
    tpb = _NW // B
    chunk_q = min(qpt, 65536 // n)
    while qpt % chunk_q:
        chunk_q -= 1
    n_chunks = qpt // chunk_q

    mesh = plsc.VectorSubcoreMesh(core_axis_name="c", subcore_axis_name="s")

    @functools.partial(
        pl.kernel,
        out_type=jax.ShapeDtypeStruct((Q * k,), jnp.int32),
        mesh=mesh,
        scratch_types=[
            pltpu.VMEM((chunk_q, n), jnp.int32),
            pltpu.VMEM((qpt,), jnp.int32),
            pltpu.VMEM((qpt * k + 16,), jnp.int32),
        ],
        compiler_params=pltpu.CompilerParams(use_tc_tiling_on_sc=False,
                                             needs_layout_passes=False),
    )
    def compact(d_hbm, v_hbm, out_hbm, d_v, v_v, out_v, ):
        wid = lax.axis_index("s") * _NC + lax.axis_index("c")
        base_q = wid * qpt
        boff = (wid // tpb) * n
        pltpu.sync_copy(v_hbm.at[pl.ds(base_q, qpt)], v_v)
        lane = jax.lax.iota(jnp.int32, 16)

        def chunk_body(c, carry):
            q0 = c * chunk_q
            pltpu.sync_copy(d_hbm.at[pl.ds(base_q + q0, chunk_q)], d_v)

            def q_body(q, carry2):
                vs = plsc.load_gather(v_v, [jnp.full((16,), 0, jnp.int32) + (q0 + q)])

                def j_body(j, cursor):
                    v = d_v[q, pl.ds(j * 16, 16)]
                    mask = v <= vs
                    cnt = plsc.cumsum(mask.astype(jnp.int32))
                    gpos = cnt + (cursor - 1 + (q0 + q) * k)
                    val = lane + (j * 16 + boff)
                    plsc.store_scatter(out_v, [gpos], val, mask=mask)
                    return cursor + jnp.sum(mask.astype(jnp.int32))

                lax.fori_loop(0, n // 16, j_body, jnp.int32(0), unroll=False)
                return carry2

            lax.fori_loop(0, chunk_q, q_body, 0, unroll=False)
            return carry

        lax.fori_loop(0, n_chunks, chunk_body, 0, unroll=False)
        pltpu.sync_copy(out_v.at[pl.ds(0, qpt * k)],
                        out_hbm.at[pl.ds(base_q * k, qpt * k)])

    return compact


def _knn_idx_flat(pos_q, pos_p, k):
    B, M, _ = pos_q.shape
    n = pos_p.shape[1]
    d_i32, vstar = _make_knn_search(B, M, n, k)(pos_q, pos_p)
    out = _make_knn_compact(B, M, n, k)(d_i32.reshape(B * M, n),
                                        vstar.reshape(B * M))
    return out


def kernel(pos, knn_idx, knn_idx_l, params):
    B, N, _ = pos.shape
    BN = B * N

    trans, pos_t = _qstn_pallas(pos, params["qstn"])
    pos = pos_t

    boffs = (jnp.arange(B, dtype=jnp.int32) * N)[:, None, None]
    idx_s = (knn_idx.astype(jnp.int32) + boffs)
    idx_s = jnp.concatenate([idx_s, idx_s], axis=-1)
    idx_l = (knn_idx_l.astype(jnp.int32) + boffs) + BN
    idx_lfe = jnp.concatenate(
        [idx_s.reshape(-1), idx_l.reshape(-1)], axis=0)

    def wcat(wb):
        w, b = wb
        C = w.shape[0] // 2
        return (jnp.concatenate([w[:C] - w[C:], w[C:]], axis=1),
                jnp.concatenate([b, jnp.zeros((24,), b.dtype)]).reshape(1, 48))

    NUM_OUT = [512, 256, 128, 64]
    KNN_H1, KNN_H2 = 32, 16

    idx1 = _knn_idx_flat(pos[:, :NUM_OUT[0]], pos, KNN_H1)
    idx2 = _knn_idx_flat(pos[:, :NUM_OUT[1]], pos[:, :NUM_OUT[0]], KNN_H1)
    idx3 = _knn_idx_flat(pos[:, :NUM_OUT[2]], pos[:, :NUM_OUT[1]], KNN_H2)
    idx4 = _knn_idx_flat(pos[:, :NUM_OUT[2]], pos[:, :NUM_OUT[2]], KNN_H2)

    x1 = pos.reshape(BN, 3)
    x2 = x1
    a1 = a2 = gm1 = gm2 = None
    for lvl in range(4):
        wc1, bc1 = wcat(params["enc1"][lvl])
        wc2, bc2 = wcat(params["enc2"][lvl])
        x1, x2, a1, a2, table = _lfe_level_pallas(
            x1, x2, a1, a2, gm1, gm2, wc1, bc1, wc2, bc2)
        gmax = _gather_max(table, idx_lfe, 32)
        gm1, gm2 = gmax[:BN], gmax[BN:]

    def b2d(b):
        return b.reshape(1, -1)

    y = _att_c12_pallas(x1, x2, a1, a2, gm1, gm2,
                        params["att"][0], b2d(params["att"][1]),
                        params["c1"][0], b2d(params["c1"][1]),
                        params["c2"][0], b2d(params["c2"][1]))

    def hier(y2d, idxf, m, p, g_prev, nf, K):
        agg = _gather_max(y2d, idxf, K)
        Nsrc = y2d.shape[0] // B
        yc = y2d.reshape(B, Nsrc, 256)[:, :m].reshape(B * m, 256)
        return _hier_pallas(yc, agg, g_prev,
                            p[0][0], b2d(p[0][1]), p[1][0], b2d(p[1][1]),
                            nf, B)

    y, g1 = hier(y, idx1, NUM_OUT[0], params["s1"], None, 1, KNN_H1)
    y, g2 = hier(y, idx2, NUM_OUT[1], params["s2"], g1, 2, KNN_H1)
    y, g3 = hier(y, idx3, NUM_OUT[2], params["s3"], g2, 1, KNN_H2)
    y, g4 = hier(y, idx4, NUM_OUT[2], params["s4"], g3, 2, KNN_H2)

    y_out, patch_global = _final_pallas(
        y, g1, g2, g3, g4,
        params["c3"][0], b2d(params["c3"][1]),
        params["c4"][0], b2d(params["c4"][1]),
        params["cg"][0], b2d(params["cg"][1]),
        params["mlp"][0][0], b2d(params["mlp"][0][1]),
        params["mlp"][1][0], b2d(params["mlp"][1][1]), B)

    y_out = jnp.transpose(y_out.reshape(B, NUM_OUT[2], 128), (0, 2, 1))
    return (y_out, trans, pos, patch_global)


# speedup vs baseline: 1.1275x; 1.0001x over previous
import functools
import jax
import jax.numpy as jnp
from jax import lax
from jax.experimental import pallas as pl
from jax.experimental.pallas import tpu as pltpu
from jax.experimental.pallas import tpu_sc as plsc

_NC, _NS = 2, 16
_NW = _NC * _NS


@functools.lru_cache(maxsize=None)
def _make_gather_max(R, D, Q, K):
    assert D % 16 == 0
    qpw = Q // _NW
    assert qpw * _NW == Q
    NB = 2
    tile_q = max(1, min(qpw, (65536 // NB) // (K * D)))
    while qpw % tile_q:
        tile_q -= 1
    n_sub = qpw // tile_q
    while n_sub % NB:
        assert tile_q % 2 == 0
        tile_q //= 2
        n_sub = qpw // tile_q

    mesh = plsc.VectorSubcoreMesh(core_axis_name="c", subcore_axis_name="s")

    @functools.partial(
        pl.kernel,
        out_type=jax.ShapeDtypeStruct((Q, D), jnp.float32),
        mesh=mesh,
        scratch_types=[
            pltpu.VMEM((qpw * K,), jnp.int32),
        ] + [pltpu.VMEM((tile_q * K, D), jnp.float32) for _ in range(NB)] + [
            pltpu.VMEM((qpw, D), jnp.float32),
        ] + [pltpu.SemaphoreType.DMA for _ in range(NB)],
        compiler_params=pltpu.CompilerParams(use_tc_tiling_on_sc=False),
    )
    def gather_max(table_hbm, idx_hbm, out_hbm, idx_v, *rest):
        bufs = rest[:NB]
        out_v = rest[NB]
        sems = rest[NB + 1:]
        wid = lax.axis_index("s") * _NC + lax.axis_index("c")
        base_q = wid * qpw
        pltpu.sync_copy(idx_hbm.at[pl.ds(base_q * K, qpw * K)], idx_v)

        def start(s, b):
            pltpu.async_copy(
                table_hbm.at[idx_v.at[pl.ds(s * (tile_q * K), tile_q * K)]],
                bufs[b], sems[b])

        def wait(b):
            pltpu.make_async_copy(
                table_hbm.at[idx_v.at[pl.ds(0, tile_q * K)]],
                bufs[b], sems[b]).wait()

        def compute(s, b):
            rows = bufs[b]

            def qbody(q, c2):
                for c in range(D // 16):
                    sl = pl.ds(c * 16, 16)
                    acc = rows[q * K, sl]
                    for k in range(1, K):
                        acc = jnp.maximum(acc, rows[q * K + k, sl])
                    out_v[s * tile_q + q, sl] = acc
                return c2

            lax.fori_loop(0, tile_q, qbody, 0, unroll=False)

        for b in range(NB - 1):
            start(b, b)

        def group(i, carry):
            s0 = i * NB
            for b in range(NB):
                start_s = s0 + b + (NB - 1)

                @pl.when(start_s < n_sub)
                def _(start_s=start_s, b=b):
                    start(start_s, (b + NB - 1) % NB)

                wait(b)
                compute(s0 + b, b)
            return carry

        lax.fori_loop(0, n_sub // NB, group, 0, unroll=False)
        pltpu.sync_copy(out_v, out_hbm.at[pl.ds(base_q, qpw)])

    return gather_max


def _gather_max(table, idx_flat, K):
    R, D = table.shape
    Q = idx_flat.shape[0] // K
    return _make_gather_max(R, D, Q, K)(table, idx_flat)


def _relu(x):
    return jnp.maximum(x, 0.0)


def _mm(a, w):
    return jnp.dot(a, w)


def _mmh(a, w):
    return jnp.dot(a, w, precision=jax.lax.Precision.HIGHEST)


def _qstn_pallas(pos, p):
    B, N, _ = pos.shape

    def mm(a, w):
        return jnp.dot(a, w, precision=jax.lax.Precision.HIGHEST)

    def body(pos_ref, w0, b0, w1, b1, w2, b2, w3, b3, w4, b4, w5, b5,
             t_ref, pt_ref):
        x = pos_ref[0]
        h = _relu(mm(x, w0[...]) + b0[...])
        h = _relu(mm(h, w1[...]) + b1[...])
        h = _relu(mm(h, w2[...]) + b2[...])
        v = jnp.max(h, axis=0, keepdims=True)
        v = _relu(mm(v, w3[...]) + b3[...])
        v = _relu(mm(v, w4[...]) + b4[...])
        q = mm(v, w5[...]) + b5[...]
        w, qx, qy, qz = (q[:, 0:1] + 1.0, q[:, 1:2], q[:, 2:3], q[:, 3:4])
        rn = jax.lax.rsqrt(w * w + qx * qx + qy * qy + qz * qz)
        w, qx, qy, qz = w * rn, qx * rn, qy * rn, qz * rn
        r = [1 - 2 * (qy * qy + qz * qz), 2 * (qx * qy - w * qz), 2 * (qx * qz + w * qy),
             2 * (qx * qy + w * qz), 1 - 2 * (qx * qx + qz * qz), 2 * (qy * qz - w * qx),
             2 * (qx * qz - w * qy), 2 * (qy * qz + w * qx), 1 - 2 * (qx * qx + qy * qy)]
        t_ref[0] = jnp.concatenate(r, axis=1)
        px, py, pz = x[:, 0:1], x[:, 1:2], x[:, 2:3]
        cols = [px * r[0] + py * r[3] + pz * r[6],
                px * r[1] + py * r[4] + pz * r[7],
                px * r[2] + py * r[5] + pz * r[8]]
        pt_ref[0] = jnp.concatenate(cols, axis=1)

    wspecs = []
    wargs = []
    for (w, b) in p:
        wspecs += [pl.BlockSpec(w.shape, lambda bb: (0, 0)),
                   pl.BlockSpec((1,) + b.shape, lambda bb: (0, 0))]
        wargs += [w, b.reshape(1, -1)]
    trans9, pos_t = pl.pallas_call(
        body,
        grid=(B,),
        in_specs=[pl.BlockSpec((1, N, 3), lambda bb: (bb, 0, 0))] + wspecs,
        out_specs=[pl.BlockSpec((1, 1, 9), lambda bb: (bb, 0, 0)),
                   pl.BlockSpec((1, N, 3), lambda bb: (bb, 0, 0))],
        out_shape=[jax.ShapeDtypeStruct((B, 1, 9), jnp.float32),
                   jax.ShapeDtypeStruct((B, N, 3), jnp.float32)],
    )(pos, *wargs)
    return trans9.reshape(B, 3, 3), pos_t


def _lfe_level_pallas(x1, x2, a1, a2, gm1, gm2, wc1, bc1, wc2, bc2):
    BN = x1.shape[0]

    def post(x, a, gm):
        if a is None:
            return x
        return jnp.concatenate([x, _relu(a + gm[:, :24])], axis=1)

    def post2(x, a, gm):
        if a is None:
            return x
        g = jnp.max(gm.reshape(gm.shape[0] // 2, 2, 32), axis=1)
        return jnp.concatenate([x, _relu(a + g[:, :24])], axis=1)

    def body(*refs):
        if a1 is None:
            x1r, x2r, wc1r, bc1r, wc2r, bc2r, x1o, x2o, a1o, a2o, tabo = refs
            x1n = x1r[...]
            x2n = x2r[...]
        else:
            (x1r, x2r, a1r, a2r, g1r, g2r, wc1r, bc1r, wc2r, bc2r,
             x1o, x2o, a1o, a2o, tabo) = refs
            x1n = post(x1r[...], a1r[...], g1r[...])
            x2n = post(x2r[...], a2r[...], g2r[...])
        R = x1n.shape[0]
        ab1 = _mm(x1n, wc1r[...]) + bc1r[...]
        ab2 = _mm(x2n, wc2r[...]) + bc2r[...]
        x1o[...] = x1n
        x2o[...] = x2n
        a1o[...] = ab1[:, :24]
        a2o[...] = ab2[:, :24]
        z = jnp.zeros((R, 8), jnp.float32)
        tabo[0] = jnp.concatenate([ab1[:, 24:], z], axis=1)
        tabo[1] = jnp.concatenate([ab2[:, 24:], z], axis=1)

    C = x1.shape[1] + (0 if a1 is None else 24)
    G = 2
    R = BN // G

    def rows(a):
        return pl.BlockSpec((R, a.shape[1]), lambda b: (b, 0))

    def full(a):
        return pl.BlockSpec(a.shape, lambda b: (0, 0))

    if a1 is None:
        args = [x1, x2, wc1, bc1, wc2, bc2]
        in_specs = [rows(x1), rows(x2), full(wc1), full(bc1), full(wc2),
                    full(bc2)]
    else:
        args = [x1, x2, a1, a2, gm1, gm2, wc1, bc1, wc2, bc2]
        in_specs = [rows(x1), rows(x2), rows(a1), rows(a2), rows(gm1),
                    rows(gm2), full(wc1), full(bc1), full(wc2), full(bc2)]
    x1n, x2n, a1n, a2n, tab = pl.pallas_call(
        body,
        grid=(G,),
        in_specs=in_specs,
        out_specs=[pl.BlockSpec((R, C), lambda b: (b, 0)),
                   pl.BlockSpec((R, C), lambda b: (b, 0)),
                   pl.BlockSpec((R, 24), lambda b: (b, 0)),
                   pl.BlockSpec((R, 24), lambda b: (b, 0)),
                   pl.BlockSpec((2, R, 32), lambda b: (0, b, 0))],
        out_shape=[jax.ShapeDtypeStruct((BN, C), jnp.float32),
                   jax.ShapeDtypeStruct((BN, C), jnp.float32),
                   jax.ShapeDtypeStruct((BN, 24), jnp.float32),
                   jax.ShapeDtypeStruct((BN, 24), jnp.float32),
                   jax.ShapeDtypeStruct((2, BN, 32), jnp.float32)],
    )(*args)
    return x1n, x2n, a1n, a2n, tab.reshape(2 * BN, 32)


def _att_c12_pallas(x1, x2, a1, a2, gm1, gm2, watt, batt, wc1, bc1, wc2, bc2):
    BN = x1.shape[0]

    def body(x1r, x2r, a1r, a2r, g1r, g2r, war, bar, w1r, b1r, w2r, b2r, yo):
        y1 = jnp.concatenate([x1r[...], _relu(a1r[...] + g1r[...][:, :24])], axis=1)
        y2 = jnp.concatenate([x2r[...], _relu(a2r[...] + g2r[...][:, :24])], axis=1)
        z = _mmh(y1 + y2, war[...]) + bar[...]
        s = 1.0 / (1.0 + jnp.exp(-z))
        y = s * y1 + (1.0 - s) * y2
        y = _relu(_mmh(y, w1r[...]) + b1r[...])
        yo[...] = _relu(_mmh(y, w2r[...]) + b2r[...])

    G = 2
    R = BN // G

    def rows(c):
        return pl.BlockSpec((R, c), lambda b: (b, 0))

    def full(a):
        return pl.BlockSpec(a.shape, lambda b: (0, 0))

    return pl.pallas_call(
        body,
        grid=(G,),
        in_specs=[rows(x1.shape[1]), rows(x2.shape[1]), rows(24), rows(24),
                  rows(32), rows(32), full(watt), full(batt), full(wc1),
                  full(bc1), full(wc2), full(bc2)],
        out_specs=rows(256),
        out_shape=jax.ShapeDtypeStruct((BN, 256), jnp.float32),
    )(x1, x2, a1, a2, gm1, gm2, watt, batt, wc1, bc1, wc2, bc2)


def _hier_pallas(yc, agg, gprev, w1, b1, w2, b2, nf, B):
    Qm = yc.shape[0]
    m = Qm // B

    def body(*refs):
        if gprev is None:
            ycr, aggr, w1r, b1r, w2r, b2r, yo, go = refs
        else:
            ycr, aggr, gpr, w1r, b1r, w2r, b2r, yo, go = refs
        ycv = ycr[...]
        a = aggr[...]
        if nf != 1:
            a = a - ycv
        f = jnp.concatenate([ycv, a], axis=1)
        if gprev is not None:
            gb = jnp.broadcast_to(gpr[...][:, None, :], (B, m, 128))
            f = jnp.concatenate([f, gb.reshape(Qm, 128)], axis=1)
        y_new = _relu(_mmh(f, w1r[...]) + b1r[...])
        yo[...] = y_new
        ymax = jnp.max(y_new.reshape(B, m, 256), axis=1)
        go[...] = _relu(_mmh(ymax, w2r[...]) + b2r[...])

    args = [yc, agg] + ([] if gprev is None else [gprev]) + [w1, b1, w2, b2]
    return pl.pallas_call(
        body,
        out_shape=[jax.ShapeDtypeStruct((Qm, 256), jnp.float32),
                   jax.ShapeDtypeStruct((B, 128), jnp.float32)],
    )(*args)


def _final_pallas(y, g1, g2, g3, g4, wc3, bc3, wc4, bc4, wcg, bcg,
                  wm1, bm1, wm2, bm2, B):
    Qm = y.shape[0]
    m = Qm // B

    def body(yr, g1r, g2r, g3r, g4r, w3r, b3r, w4r, b4r, wgr, bgr,
             wm1r, bm1r, wm2r, bm2r, yo, po):
        yv = yr[...]
        t = _relu(_mmh(yv, w3r[...]) + b3r[...]) + yv
        t = _relu(_mmh(t, w4r[...]) + b4r[...])
        yo[...] = t
        t3 = t.reshape(B, m, 128)[:, :64]
        t2 = t3.reshape(B * 64, 128)
        yg = _relu(_mmh(t2, wgr[...]) + bgr[...]) + t2
        y_g = jnp.max(yg.reshape(B, 64, 128), axis=1)
        h = jnp.concatenate([g1r[...], g2r[...], g3r[...], g4r[...], y_g],
                            axis=1)
        h = _relu(_mmh(h, wm1r[...]) + bm1r[...])
        po[...] = _relu(_mmh(h, wm2r[...]) + bm2r[...])

    return pl.pallas_call(
        body,
        out_shape=[jax.ShapeDtypeStruct((Qm, 128), jnp.float32),
                   jax.ShapeDtypeStruct((B, 128), jnp.float32)],
    )(y, g1, g2, g3, g4, wc3, bc3, wc4, bc4, wcg, bcg, wm1, bm1, wm2, bm2)


@functools.lru_cache(maxsize=None)
def _make_knn_search(B, M, n, k):
    def body(q_ref, p_ref, d_ref, v_ref):
        q = q_ref[0]
        p = p_ref[0]
        dx = q[:, 0:1] - p[:, 0][None, :]
        dy = q[:, 1:2] - p[:, 1][None, :]
        dz = q[:, 2:3] - p[:, 2][None, :]
        d = dx * dx + dy * dy + dz * dz
        ri = lax.broadcasted_iota(jnp.int32, (M, n), 0)
        ci = lax.broadcasted_iota(jnp.int32, (M, n), 1)
        d = jnp.where(ri == ci, jnp.float32(jnp.inf), d)
        di = lax.bitcast_convert_type(d, jnp.int32)

        def it(_, lohi):
            lo, hi = lohi
            mid = lo + (hi - lo) // 2
            cnt = jnp.sum((di <= mid).astype(jnp.float32), axis=1, keepdims=True)
            ge = cnt >= k
            return jnp.where(ge, lo, mid + 1), jnp.where(ge, mid, hi)

        lo0 = jnp.zeros((M, 1), jnp.int32)
        hi0 = jnp.full((M, 1), 0x7F800000, jnp.int32)
        lo, hi = lax.fori_loop(0, 31, it, (lo0, hi0))
        d_ref[0] = di
        v_ref[0, 0] = hi[:, 0]

    return pl.pallas_call(
        body,
        grid=(B,),
        in_specs=[
            pl.BlockSpec((1, M, 3), lambda b: (b, 0, 0)),
            pl.BlockSpec((1, n, 3), lambda b: (b, 0, 0)),
        ],
        out_specs=[
            pl.BlockSpec((1, M, n), lambda b: (b, 0, 0)),
            pl.BlockSpec((1, 1, M), lambda b: (b, 0, 0)),
        ],
        out_shape=[
            jax.ShapeDtypeStruct((B, M, n), jnp.int32),
            jax.ShapeDtypeStruct((B, 1, M), jnp.int32),
        ],
    )


@functools.lru_cache(maxsize=None)
def _make_knn_compact(B, M, n, k):
    Q = B * M
    qpt = Q // _NW
    tpb = _NW // B
    chunk_q = min(qpt, 65536 // n)
    while qpt % chunk_q:
        chunk_q -= 1
    n_chunks = qpt // chunk_q

    mesh = plsc.VectorSubcoreMesh(core_axis_name="c", subcore_axis_name="s")

    @functools.partial(
        pl.kernel,
        out_type=jax.ShapeDtypeStruct((Q * k,), jnp.int32),
        mesh=mesh,
        scratch_types=[
            pltpu.VMEM((chunk_q, n), jnp.int32),
            pltpu.VMEM((qpt,), jnp.int32),
            pltpu.VMEM((qpt * k + 16,), jnp.int32),
        ],
        compiler_params=pltpu.CompilerParams(use_tc_tiling_on_sc=False,
                                             needs_layout_passes=False),
    )
    def compact(d_hbm, v_hbm, out_hbm, d_v, v_v, out_v, ):
        wid = lax.axis_index("s") * _NC + lax.axis_index("c")
        base_q = wid * qpt
        boff = (wid // tpb) * n
        pltpu.sync_copy(v_hbm.at[pl.ds(base_q, qpt)], v_v)
        lane = jax.lax.iota(jnp.int32, 16)

        def chunk_body(c, carry):
            q0 = c * chunk_q
            pltpu.sync_copy(d_hbm.at[pl.ds(base_q + q0, chunk_q)], d_v)

            def q_body(q, carry2):
                vs = plsc.load_gather(v_v, [jnp.full((16,), 0, jnp.int32) + (q0 + q)])

                def j_body(j, cursor):
                    v = d_v[q, pl.ds(j * 16, 16)]
                    mask = v <= vs
                    cnt = plsc.cumsum(mask.astype(jnp.int32))
                    gpos = cnt + (cursor - 1 + (q0 + q) * k)
                    val = lane + (j * 16 + boff)
                    plsc.store_scatter(out_v, [gpos], val, mask=mask)
                    return cursor + cnt[15]

                lax.fori_loop(0, n // 16, j_body, jnp.int32(0), unroll=False)
                return carry2

            lax.fori_loop(0, chunk_q, q_body, 0, unroll=False)
            return carry

        lax.fori_loop(0, n_chunks, chunk_body, 0, unroll=False)
        pltpu.sync_copy(out_v.at[pl.ds(0, qpt * k)],
                        out_hbm.at[pl.ds(base_q * k, qpt * k)])

    return compact


def _knn_idx_flat(pos_q, pos_p, k):
    B, M, _ = pos_q.shape
    n = pos_p.shape[1]
    d_i32, vstar = _make_knn_search(B, M, n, k)(pos_q, pos_p)
    out = _make_knn_compact(B, M, n, k)(d_i32.reshape(B * M, n),
                                        vstar.reshape(B * M))
    return out


def kernel(pos, knn_idx, knn_idx_l, params):
    B, N, _ = pos.shape
    BN = B * N

    trans, pos_t = _qstn_pallas(pos, params["qstn"])
    pos = pos_t

    boffs = (jnp.arange(B, dtype=jnp.int32) * N)[:, None, None]
    idx_s = (knn_idx.astype(jnp.int32) + boffs)
    idx_s = jnp.concatenate([idx_s, idx_s], axis=-1)
    idx_l = (knn_idx_l.astype(jnp.int32) + boffs) + BN
    idx_lfe = jnp.concatenate(
        [idx_s.reshape(-1), idx_l.reshape(-1)], axis=0)

    def wcat(wb):
        w, b = wb
        C = w.shape[0] // 2
        return (jnp.concatenate([w[:C] - w[C:], w[C:]], axis=1),
                jnp.concatenate([b, jnp.zeros((24,), b.dtype)]).reshape(1, 48))

    NUM_OUT = [512, 256, 128, 64]
    KNN_H1, KNN_H2 = 32, 16

    idx1 = _knn_idx_flat(pos[:, :NUM_OUT[0]], pos, KNN_H1)
    idx2 = _knn_idx_flat(pos[:, :NUM_OUT[1]], pos[:, :NUM_OUT[0]], KNN_H1)
    idx3 = _knn_idx_flat(pos[:, :NUM_OUT[2]], pos[:, :NUM_OUT[1]], KNN_H2)
    idx4 = _knn_idx_flat(pos[:, :NUM_OUT[2]], pos[:, :NUM_OUT[2]], KNN_H2)

    x1 = pos.reshape(BN, 3)
    x2 = x1
    a1 = a2 = gm1 = gm2 = None
    for lvl in range(4):
        wc1, bc1 = wcat(params["enc1"][lvl])
        wc2, bc2 = wcat(params["enc2"][lvl])
        x1, x2, a1, a2, table = _lfe_level_pallas(
            x1, x2, a1, a2, gm1, gm2, wc1, bc1, wc2, bc2)
        gmax = _gather_max(table, idx_lfe, 32)
        gm1, gm2 = gmax[:BN], gmax[BN:]

    def b2d(b):
        return b.reshape(1, -1)

    y = _att_c12_pallas(x1, x2, a1, a2, gm1, gm2,
                        params["att"][0], b2d(params["att"][1]),
                        params["c1"][0], b2d(params["c1"][1]),
                        params["c2"][0], b2d(params["c2"][1]))

    def hier(y2d, idxf, m, p, g_prev, nf, K):
        agg = _gather_max(y2d, idxf, K)
        Nsrc = y2d.shape[0] // B
        yc = y2d.reshape(B, Nsrc, 256)[:, :m].reshape(B * m, 256)
        return _hier_pallas(yc, agg, g_prev,
                            p[0][0], b2d(p[0][1]), p[1][0], b2d(p[1][1]),
                            nf, B)

    y, g1 = hier(y, idx1, NUM_OUT[0], params["s1"], None, 1, KNN_H1)
    y, g2 = hier(y, idx2, NUM_OUT[1], params["s2"], g1, 2, KNN_H1)
    y, g3 = hier(y, idx3, NUM_OUT[2], params["s3"], g2, 1, KNN_H2)
    y, g4 = hier(y, idx4, NUM_OUT[2], params["s4"], g3, 2, KNN_H2)

    y_out, patch_global = _final_pallas(
        y, g1, g2, g3, g4,
        params["c3"][0], b2d(params["c3"][1]),
        params["c4"][0], b2d(params["c4"][1]),
        params["cg"][0], b2d(params["cg"][1]),
        params["mlp"][0][0], b2d(params["mlp"][0][1]),
        params["mlp"][1][0], b2d(params["mlp"][1][1]), B)

    y_out = jnp.transpose(y_out.reshape(B, NUM_OUT[2], 128), (0, 2, 1))
    return (y_out, trans, pos, patch_global)
